# baseline (speedup 1.0000x reference)
import jax
import jax.numpy as jnp
import numpy as np
from jax.experimental import pallas as pl
from jax.experimental.pallas import tpu as pltpu

_N = 16384
_D = 768
_T = 2048
_B = 4
_DECAY = 0.9
_ROUNDS = 10
_THRESH = 0.5

_NB = 8
_BN = _N // _NB
_TB = 8
_BT = _T // _TB
_SCALE = np.float32(1.0 / np.sqrt(np.float32(_D)))
_R4 = _ROUNDS * _B


def _body(qs_ref, keys_ref, values_ref, pri_ref, ages_ref, vm_ref,
          wq_ref, bq_ref, wc_ref, bc_ref, out_ref, qv, p40, s0, s1, mb,
          g_scr, h_scr):
    j = pl.program_id(0)
    f32 = jnp.float32
    bf16 = jnp.bfloat16

    @pl.when(j == 0)
    def _init():
        qv[...] = jnp.zeros_like(qv)
        p40[...] = jnp.zeros_like(p40)
        s0[...] = jnp.zeros_like(s0)
        s1[...] = jnp.zeros_like(s1)
        mb[...] = jnp.full_like(mb, -1e30)

    @pl.when(j < _TB)
    def _pool():
        qv[...] += jnp.sum(qs_ref[...], axis=1)

    @pl.when(j < _NB)
    def _tables():
        log_decay = np.float32(np.log(_DECAY))
        eff0 = pri_ref[0] * jnp.exp(ages_ref[0] * log_decay)
        dpow = jnp.exp(log_decay * jax.lax.broadcasted_iota(
            jnp.int32, (_ROUNDS, 1), 0).astype(f32))
        eff_stack = dpow * eff0
        h_stack = eff_stack + 1e-8
        g_stack = h_stack * (jax.nn.sigmoid((eff_stack - _THRESH) * 10.0)
                             * vm_ref[0])
        g_scr[j] = g_stack
        h_scr[j] = h_stack

    @pl.when(j == _TB)
    def _project_q():
        q = qv[...] * (1.0 / _T)
        qv[...] = jax.lax.dot_general(
            q, wq_ref[...], (((1,), (1,)), ((), ())),
            preferred_element_type=f32) + bq_ref[...]

    @pl.when(j >= _TB)
    def _block():
        s = jax.lax.dot_general(
            qv[...].astype(bf16), keys_ref[...].astype(bf16),
            (((1,), (1,)), ((), ())), preferred_element_type=f32) * _SCALE

        g40 = jnp.repeat(g_scr[j - _TB], _B, axis=0)
        h40 = jnp.repeat(h_scr[j - _TB], _B, axis=0)

        bm = jnp.max(s, axis=1, keepdims=True)
        mb_new = jnp.maximum(mb[...], bm)
        sc = jnp.exp(mb[...] - mb_new)
        mb[...] = mb_new
        e = jnp.exp(s - mb_new)
        e40 = jnp.concatenate([e] * _ROUNDS, axis=0)
        sc40 = jnp.concatenate([sc] * _ROUNDS, axis=0)

        num40 = e40 * g40
        den40 = e40 * h40
        s1[...] = s1[...] * sc40 + jnp.sum(num40, axis=1, keepdims=True)
        s0[...] = s0[...] * sc40 + jnp.sum(den40, axis=1, keepdims=True)
        p40[...] = p40[...] * sc40 + jax.lax.dot_general(
            num40.astype(bf16), values_ref[...].astype(bf16),
            (((1,), (0,)), ((), ())), preferred_element_type=f32)

    @pl.when(j == _TB + _NB - 1)
    def _final():
        c40 = 1.0 / (s1[...] + 1e-8 * s0[...])
        wp = p40[...] * c40
        acc = jnp.zeros((_B, _D), f32)
        for r in range(_ROUNDS):
            acc = acc + wp[r * _B:(r + 1) * _B, :]
        out_ref[...] = jax.lax.dot_general(
            acc, wc_ref[...], (((1,), (1,)), ((), ())),
            preferred_element_type=f32) * (1.0 / _ROUNDS) + bc_ref[...]


@jax.jit
def kernel(query_states, keys, values, priorities, Wq, bq, Wc, bc, ages,
           valid_mask):
    B, T, D = query_states.shape

    pri = priorities.reshape(_NB, 1, _BN)
    ages_f = ages.astype(jnp.float32).reshape(_NB, 1, _BN)
    vm = valid_mask.astype(jnp.float32).reshape(_NB, 1, _BN)
    bq2 = bq.reshape(1, D)
    bc2 = bc.reshape(1, D)

    qs_map = lambda j: (0, jnp.minimum(j, _TB - 1), 0)
    nb_map = lambda j: (jnp.maximum(j - _TB, 0), 0)
    sm_map = lambda j: (jnp.minimum(j, _NB - 1), 0, 0)

    out = pl.pallas_call(
        _body,
        grid=(_TB + _NB,),
        in_specs=[
            pl.BlockSpec((B, _BT, D), qs_map),
            pl.BlockSpec((_BN, _D), nb_map),
            pl.BlockSpec((_BN, _D), nb_map),
            pl.BlockSpec((1, 1, _BN), sm_map),
            pl.BlockSpec((1, 1, _BN), sm_map),
            pl.BlockSpec((1, 1, _BN), sm_map),
            pl.BlockSpec((_D, _D), lambda j: (0, 0)),
            pl.BlockSpec((1, _D), lambda j: (0, 0)),
            pl.BlockSpec((_D, _D), lambda j: (0, 0)),
            pl.BlockSpec((1, _D), lambda j: (0, 0)),
        ],
        out_specs=pl.BlockSpec((B, D), lambda j: (0, 0)),
        out_shape=jax.ShapeDtypeStruct((B, D), jnp.float32),
        scratch_shapes=[
            pltpu.VMEM((B, _D), jnp.float32),
            pltpu.VMEM((_R4, _D), jnp.float32),
            pltpu.VMEM((_R4, 1), jnp.float32),
            pltpu.VMEM((_R4, 1), jnp.float32),
            pltpu.VMEM((B, 1), jnp.float32),
            pltpu.VMEM((_NB, _ROUNDS, _BN), jnp.float32),
            pltpu.VMEM((_NB, _ROUNDS, _BN), jnp.float32),
        ],
    )(query_states, keys, values, pri, ages_f, vm, Wq, bq2, Wc, bc2)
    return out


# speedup vs baseline: 1.0205x; 1.0205x over previous
import jax
import jax.numpy as jnp
import numpy as np
from jax.experimental import pallas as pl
from jax.experimental.pallas import tpu as pltpu

_N = 16384
_D = 768
_T = 2048
_B = 4
_DECAY = 0.9
_ROUNDS = 10
_THRESH = 0.5

_NB = 8
_BN = _N // _NB
_TB = 8
_BT = _T // _TB
_SCALE = np.float32(1.0 / np.sqrt(np.float32(_D)))
_R4 = _ROUNDS * _B


def _body(qs_ref, keys_ref, values_ref, pri_ref, ages_ref, vm_ref,
          wq_ref, bq_ref, wc_ref, bc_ref, out_ref, qv, p40, s0, s1, mb):
    j = pl.program_id(0)
    f32 = jnp.float32
    bf16 = jnp.bfloat16

    @pl.when(j == 0)
    def _init():
        qv[...] = jnp.zeros_like(qv)
        p40[...] = jnp.zeros_like(p40)
        s0[...] = jnp.zeros_like(s0)
        s1[...] = jnp.zeros_like(s1)
        mb[...] = jnp.full_like(mb, -1e30)

    @pl.when(j < _TB)
    def _pool():
        qv[...] += jnp.sum(qs_ref[...], axis=1)

    @pl.when(j == _TB)
    def _project_q():
        q = qv[...] * (1.0 / _T)
        qv[...] = jax.lax.dot_general(
            q, wq_ref[...], (((1,), (1,)), ((), ())),
            preferred_element_type=f32) + bq_ref[...]

    @pl.when(j >= _TB)
    def _block():
        s = jax.lax.dot_general(
            qv[...].astype(bf16), keys_ref[...].astype(bf16),
            (((1,), (1,)), ((), ())), preferred_element_type=f32) * _SCALE

        log_decay = np.float32(np.log(_DECAY))
        eff0 = pri_ref[0] * jnp.exp(ages_ref[0] * log_decay)
        dpow = jnp.exp(log_decay * jax.lax.broadcasted_iota(
            jnp.int32, (_ROUNDS, 1), 0).astype(f32))
        eff_stack = dpow * eff0
        h_stack = eff_stack + 1e-8
        g_stack = h_stack * (jax.nn.sigmoid((eff_stack - _THRESH) * 10.0)
                             * vm_ref[0])
        g40 = jnp.repeat(g_stack, _B, axis=0)
        h40 = jnp.repeat(h_stack, _B, axis=0)

        bm = jnp.max(s, axis=1, keepdims=True)
        mb_new = jnp.maximum(mb[...], bm)
        sc = jnp.exp(mb[...] - mb_new)
        mb[...] = mb_new
        e = jnp.exp(s - mb_new)
        e40 = jnp.concatenate([e] * _ROUNDS, axis=0)
        sc40 = jnp.concatenate([sc] * _ROUNDS, axis=0)

        num40 = e40 * g40
        den40 = e40 * h40
        s1[...] = s1[...] * sc40 + jnp.sum(num40, axis=1, keepdims=True)
        s0[...] = s0[...] * sc40 + jnp.sum(den40, axis=1, keepdims=True)
        p40[...] = p40[...] * sc40 + jax.lax.dot_general(
            num40.astype(bf16), values_ref[...].astype(bf16),
            (((1,), (0,)), ((), ())), preferred_element_type=f32)

    @pl.when(j == _TB + _NB - 1)
    def _final():
        c40 = 1.0 / (s1[...] + 1e-8 * s0[...])
        wp = p40[...] * c40
        acc = jnp.zeros((_B, _D), f32)
        for r in range(_ROUNDS):
            acc = acc + wp[r * _B:(r + 1) * _B, :]
        out_ref[...] = jax.lax.dot_general(
            acc, wc_ref[...], (((1,), (1,)), ((), ())),
            preferred_element_type=f32) * (1.0 / _ROUNDS) + bc_ref[...]


@jax.jit
def kernel(query_states, keys, values, priorities, Wq, bq, Wc, bc, ages,
           valid_mask):
    B, T, D = query_states.shape

    pri = priorities.reshape(_NB, 1, _BN)
    ages_f = ages.astype(jnp.float32).reshape(_NB, 1, _BN)
    vm = valid_mask.astype(jnp.float32).reshape(_NB, 1, _BN)
    bq2 = bq.reshape(1, D)
    bc2 = bc.reshape(1, D)

    qs_map = lambda j: (0, jnp.minimum(j, _TB - 1), 0)
    nb_map = lambda j: (jnp.maximum(j - _TB, 0), 0)
    sm_map = lambda j: (jnp.maximum(j - _TB, 0), 0, 0)

    out = pl.pallas_call(
        _body,
        grid=(_TB + _NB,),
        in_specs=[
            pl.BlockSpec((B, _BT, D), qs_map),
            pl.BlockSpec((_BN, _D), nb_map),
            pl.BlockSpec((_BN, _D), nb_map),
            pl.BlockSpec((1, 1, _BN), sm_map),
            pl.BlockSpec((1, 1, _BN), sm_map),
            pl.BlockSpec((1, 1, _BN), sm_map),
            pl.BlockSpec((_D, _D), lambda j: (0, 0)),
            pl.BlockSpec((1, _D), lambda j: (0, 0)),
            pl.BlockSpec((_D, _D), lambda j: (0, 0)),
            pl.BlockSpec((1, _D), lambda j: (0, 0)),
        ],
        out_specs=pl.BlockSpec((B, D), lambda j: (0, 0)),
        out_shape=jax.ShapeDtypeStruct((B, D), jnp.float32),
        scratch_shapes=[
            pltpu.VMEM((B, _D), jnp.float32),
            pltpu.VMEM((_R4, _D), jnp.float32),
            pltpu.VMEM((_R4, 1), jnp.float32),
            pltpu.VMEM((_R4, 1), jnp.float32),
            pltpu.VMEM((B, 1), jnp.float32),
        ],
    )(query_states, keys, values, pri, ages_f, vm, Wq, bq2, Wc, bc2)
    return out
